# baseline (speedup 1.0000x reference)
import functools

import jax
import jax.numpy as jnp
from jax import lax
from jax.experimental import pallas as pl
from jax.experimental.pallas import tpu as pltpu
from jax.experimental.pallas import tpu_sc as plsc

N = 16384
M = 4096
NC = 2
NS = 16
NW = NC * NS
R = N // NW
BR = 8
NBLK = R // BR
JCH = M // 16


def _sc_body(a_hbm, x_hbm, out_hbm, x_v, a0, a1, y_v, sem0, sem1):
    wid = lax.axis_index("s") * NC + lax.axis_index("c")
    base = wid * R

    pltpu.sync_copy(x_hbm, x_v)
    pltpu.async_copy(a_hbm.at[pl.ds(base, BR)], a0, sem0)
    pltpu.async_copy(a_hbm.at[pl.ds(base + BR, BR)], a1, sem1)

    def compute8(a_ref, i, half):
        def jbody(j, accs):
            xj = x_v[pl.ds(j * 16, 16)]
            return tuple(accs[r] + a_ref[r, pl.ds(j * 16, 16)] * xj
                         for r in range(BR))
        accs = lax.fori_loop(
            0, JCH, jbody,
            tuple(jnp.zeros((16,), jnp.float32) for _ in range(BR)))
        row0 = (i * 2 + half) * BR
        for r in range(BR):
            y_v[pl.ds((row0 + r) * 16, 16)] = accs[r]

    def outer(i, carry):
        blk = i * 2
        pltpu.make_async_copy(a_hbm.at[pl.ds(base, BR)], a0, sem0).wait()
        compute8(a0, i, 0)

        @pl.when(blk + 2 < NBLK)
        def _():
            pltpu.async_copy(
                a_hbm.at[pl.ds(base + (blk + 2) * BR, BR)], a0, sem0)

        pltpu.make_async_copy(a_hbm.at[pl.ds(base, BR)], a1, sem1).wait()
        compute8(a1, i, 1)

        @pl.when(blk + 3 < NBLK)
        def _():
            pltpu.async_copy(
                a_hbm.at[pl.ds(base + (blk + 3) * BR, BR)], a1, sem1)

        return carry

    lax.fori_loop(0, NBLK // 2, outer, 0)
    pltpu.sync_copy(y_v, out_hbm.at[pl.ds(wid * R * 16, R * 16)])


@functools.partial(
    pl.kernel,
    out_type=jax.ShapeDtypeStruct((N * 16,), jnp.float32),
    mesh=plsc.VectorSubcoreMesh(core_axis_name="c", subcore_axis_name="s"),
    scratch_types=[
        pltpu.VMEM((M,), jnp.float32),
        pltpu.VMEM((BR, M), jnp.float32),
        pltpu.VMEM((BR, M), jnp.float32),
        pltpu.VMEM((R * 16,), jnp.float32),
        pltpu.SemaphoreType.DMA,
        pltpu.SemaphoreType.DMA,
    ],
)
def _sc_mv_partial(a_hbm, x_hbm, out_hbm, *rest):
    _sc_body(a_hbm, x_hbm, out_hbm, *rest)


def _fold_body(p_ref, o_ref):
    o_ref[...] = jnp.sum(p_ref[...], axis=-1)


_fold = pl.pallas_call(
    _fold_body,
    out_shape=jax.ShapeDtypeStruct((N,), jnp.float32),
)


TC_BLK = 1024


def _tc_mv_body(a_ref, x_ref, o_ref):
    o_ref[...] = jax.lax.dot_general(
        a_ref[...], x_ref[...],
        (((1,), (0,)), ((), ())),
        preferred_element_type=jnp.float32)


def _tc_mv(A, x, rows):
    return pl.pallas_call(
        _tc_mv_body,
        grid=(rows // TC_BLK,),
        in_specs=[
            pl.BlockSpec((TC_BLK, M), lambda i: (i, 0)),
            pl.BlockSpec((M,), lambda i: (0,)),
        ],
        out_specs=pl.BlockSpec((TC_BLK,), lambda i: (i,)),
        out_shape=jax.ShapeDtypeStruct((rows,), jnp.float32),
    )(A, x)


def kernel(A, x):
    part = _sc_mv_partial(A, x)
    return _fold(part.reshape(N, 16))


# speedup vs baseline: 1.3756x; 1.3756x over previous
import functools

import jax
import jax.numpy as jnp
from jax import lax
from jax.experimental import pallas as pl
from jax.experimental.pallas import tpu as pltpu
from jax.experimental.pallas import tpu_sc as plsc

N = 16384
M = 4096
NC = 2
NS = 16
NW = NC * NS

N_SC = 4096
N_TC = N - N_SC
R = N_SC // NW
BR = 8
NBLK = R // BR
JCH = M // 16

TC_BLK = 1024


def _sc_body(a_hbm, x_hbm, out_hbm, x_v, a0, a1, y_v, sem0, sem1):
    wid = lax.axis_index("s") * NC + lax.axis_index("c")
    base = N_TC + wid * R

    pltpu.sync_copy(x_hbm, x_v)
    pltpu.async_copy(a_hbm.at[pl.ds(base, BR)], a0, sem0)
    pltpu.async_copy(a_hbm.at[pl.ds(base + BR, BR)], a1, sem1)

    def compute8(a_ref, i, half):
        def jbody(j, accs):
            xj = x_v[pl.ds(j * 16, 16)]
            return tuple(accs[r] + a_ref[r, pl.ds(j * 16, 16)] * xj
                         for r in range(BR))
        accs = lax.fori_loop(
            0, JCH, jbody,
            tuple(jnp.zeros((16,), jnp.float32) for _ in range(BR)))
        row0 = (i * 2 + half) * BR
        for r in range(BR):
            y_v[pl.ds((row0 + r) * 16, 16)] = accs[r]

    def outer(i, carry):
        blk = i * 2
        pltpu.make_async_copy(a_hbm.at[pl.ds(base, BR)], a0, sem0).wait()
        compute8(a0, i, 0)

        @pl.when(blk + 2 < NBLK)
        def _():
            pltpu.async_copy(
                a_hbm.at[pl.ds(base + (blk + 2) * BR, BR)], a0, sem0)

        pltpu.make_async_copy(a_hbm.at[pl.ds(base, BR)], a1, sem1).wait()
        compute8(a1, i, 1)

        @pl.when(blk + 3 < NBLK)
        def _():
            pltpu.async_copy(
                a_hbm.at[pl.ds(base + (blk + 3) * BR, BR)], a1, sem1)

        return carry

    lax.fori_loop(0, NBLK // 2, outer, 0)
    pltpu.sync_copy(y_v, out_hbm.at[pl.ds(wid * R * 16, R * 16)])


@functools.partial(
    pl.kernel,
    out_type=jax.ShapeDtypeStruct((N_SC * 16,), jnp.float32),
    mesh=plsc.VectorSubcoreMesh(core_axis_name="c", subcore_axis_name="s"),
    scratch_types=[
        pltpu.VMEM((M,), jnp.float32),
        pltpu.VMEM((BR, M), jnp.float32),
        pltpu.VMEM((BR, M), jnp.float32),
        pltpu.VMEM((R * 16,), jnp.float32),
        pltpu.SemaphoreType.DMA,
        pltpu.SemaphoreType.DMA,
    ],
)
def _sc_mv_partial(a_hbm, x_hbm, out_hbm, *rest):
    _sc_body(a_hbm, x_hbm, out_hbm, *rest)


def _fold_body(p_ref, o_ref):
    o_ref[...] = jnp.sum(p_ref[...], axis=-1)


_fold = pl.pallas_call(
    _fold_body,
    out_shape=jax.ShapeDtypeStruct((N_SC,), jnp.float32),
)


def _tc_mv_body(a_ref, x_ref, o_ref):
    o_ref[...] = jax.lax.dot_general(
        a_ref[...], x_ref[...],
        (((1,), (0,)), ((), ())),
        preferred_element_type=jnp.float32)


_tc_mv = pl.pallas_call(
    _tc_mv_body,
    grid=(N_TC // TC_BLK,),
    in_specs=[
        pl.BlockSpec((TC_BLK, M), lambda i: (i, 0)),
        pl.BlockSpec((M,), lambda i: (0,)),
    ],
    out_specs=pl.BlockSpec((TC_BLK,), lambda i: (i,)),
    out_shape=jax.ShapeDtypeStruct((N_TC,), jnp.float32),
)


def kernel(A, x):
    part = _sc_mv_partial(A, x)
    y_tc = _tc_mv(A, x)
    y_sc = _fold(part.reshape(N_SC, 16))
    return jnp.concatenate([y_tc, y_sc])


# speedup vs baseline: 1.3778x; 1.0016x over previous
import functools

import jax
import jax.numpy as jnp
from jax import lax
from jax.experimental import pallas as pl
from jax.experimental.pallas import tpu as pltpu
from jax.experimental.pallas import tpu_sc as plsc

N = 16384
M = 4096
NC = 2
NS = 16
NW = NC * NS

N_SC = 4096
N_TC = N - N_SC
R = N_SC // NW
BR = 8
NBLK = R // BR
JCH = M // 16

TC_BLK = 1024


def _sc_body(a_hbm, x_hbm, out_hbm, x_v, a0, a1, y_v, sem0, sem1):
    wid = lax.axis_index("s") * NC + lax.axis_index("c")
    base = N_TC + wid * R

    pltpu.sync_copy(x_hbm, x_v)
    pltpu.async_copy(a_hbm.at[pl.ds(base, BR)], a0, sem0)
    pltpu.async_copy(a_hbm.at[pl.ds(base + BR, BR)], a1, sem1)

    def compute8(a_ref, i, half):
        def jbody(j, accs):
            xj = x_v[pl.ds(j * 16, 16)]
            return tuple(accs[r] + a_ref[r, pl.ds(j * 16, 16)] * xj
                         for r in range(BR))
        accs = lax.fori_loop(
            0, JCH, jbody,
            tuple(jnp.zeros((16,), jnp.float32) for _ in range(BR)),
            unroll=8)
        row0 = (i * 2 + half) * BR
        for r in range(BR):
            y_v[pl.ds((row0 + r) * 16, 16)] = accs[r]

    def outer(i, carry):
        blk = i * 2
        pltpu.make_async_copy(a_hbm.at[pl.ds(base, BR)], a0, sem0).wait()
        compute8(a0, i, 0)

        @pl.when(blk + 2 < NBLK)
        def _():
            pltpu.async_copy(
                a_hbm.at[pl.ds(base + (blk + 2) * BR, BR)], a0, sem0)

        pltpu.make_async_copy(a_hbm.at[pl.ds(base, BR)], a1, sem1).wait()
        compute8(a1, i, 1)

        @pl.when(blk + 3 < NBLK)
        def _():
            pltpu.async_copy(
                a_hbm.at[pl.ds(base + (blk + 3) * BR, BR)], a1, sem1)

        return carry

    lax.fori_loop(0, NBLK // 2, outer, 0)
    pltpu.sync_copy(y_v, out_hbm.at[pl.ds(wid * R * 16, R * 16)])


@functools.partial(
    pl.kernel,
    out_type=jax.ShapeDtypeStruct((N_SC * 16,), jnp.float32),
    mesh=plsc.VectorSubcoreMesh(core_axis_name="c", subcore_axis_name="s"),
    scratch_types=[
        pltpu.VMEM((M,), jnp.float32),
        pltpu.VMEM((BR, M), jnp.float32),
        pltpu.VMEM((BR, M), jnp.float32),
        pltpu.VMEM((R * 16,), jnp.float32),
        pltpu.SemaphoreType.DMA,
        pltpu.SemaphoreType.DMA,
    ],
)
def _sc_mv_partial(a_hbm, x_hbm, out_hbm, *rest):
    _sc_body(a_hbm, x_hbm, out_hbm, *rest)


def _fold_body(p_ref, o_ref):
    o_ref[...] = jnp.sum(p_ref[...], axis=-1)


_fold = pl.pallas_call(
    _fold_body,
    out_shape=jax.ShapeDtypeStruct((N_SC,), jnp.float32),
)


def _tc_mv_body(a_ref, x_ref, o_ref):
    o_ref[...] = jax.lax.dot_general(
        a_ref[...], x_ref[...],
        (((1,), (0,)), ((), ())),
        preferred_element_type=jnp.float32)


_tc_mv = pl.pallas_call(
    _tc_mv_body,
    grid=(N_TC // TC_BLK,),
    in_specs=[
        pl.BlockSpec((TC_BLK, M), lambda i: (i, 0)),
        pl.BlockSpec((M,), lambda i: (0,)),
    ],
    out_specs=pl.BlockSpec((TC_BLK,), lambda i: (i,)),
    out_shape=jax.ShapeDtypeStruct((N_TC,), jnp.float32),
)


def kernel(A, x):
    part = _sc_mv_partial(A, x)
    y_tc = _tc_mv(A, x)
    y_sc = _fold(part.reshape(N_SC, 16))
    return jnp.concatenate([y_tc, y_sc])


# speedup vs baseline: 1.5739x; 1.1423x over previous
import functools

import jax
import jax.numpy as jnp
from jax import lax
from jax.experimental import pallas as pl
from jax.experimental.pallas import tpu as pltpu
from jax.experimental.pallas import tpu_sc as plsc

N = 16384
M = 4096
NC = 1
NS = 16
NW = NC * NS

N_SC = 4096
N_TC = N - N_SC
R = N_SC // NW
BR = 8
NBLK = R // BR
JCH = M // 16

TC_BLK = 512


def _sc_body(a_hbm, x_hbm, out_hbm, x_v, a0, a1, y_v, y2d, ysh, sem0, sem1):
    wid = lax.axis_index("s") * NC + lax.axis_index("c")
    base = N_TC + wid * R

    pltpu.sync_copy(x_hbm, x_v)
    pltpu.async_copy(a_hbm.at[pl.ds(base, BR)], a0, sem0)
    pltpu.async_copy(a_hbm.at[pl.ds(base + BR, BR)], a1, sem1)

    def compute8(a_ref, i, half):
        def jbody(j, accs):
            xj = x_v[pl.ds(j * 16, 16)]
            return tuple(accs[r] + a_ref[r, pl.ds(j * 16, 16)] * xj
                         for r in range(BR))
        accs = lax.fori_loop(
            0, JCH, jbody,
            tuple(jnp.zeros((16,), jnp.float32) for _ in range(BR)),
            unroll=8)
        row0 = (i * 2 + half) * BR
        for r in range(BR):
            y_v[pl.ds((row0 + r) * 16, 16)] = accs[r]

    def outer(i, carry):
        blk = i * 2
        pltpu.make_async_copy(a_hbm.at[pl.ds(base, BR)], a0, sem0).wait()
        compute8(a0, i, 0)

        @pl.when(blk + 2 < NBLK)
        def _():
            pltpu.async_copy(
                a_hbm.at[pl.ds(base + (blk + 2) * BR, BR)], a0, sem0)

        pltpu.make_async_copy(a_hbm.at[pl.ds(base, BR)], a1, sem1).wait()
        compute8(a1, i, 1)

        @pl.when(blk + 3 < NBLK)
        def _():
            pltpu.async_copy(
                a_hbm.at[pl.ds(base + (blk + 3) * BR, BR)], a1, sem1)

        return carry

    lax.fori_loop(0, NBLK // 2, outer, 0)

    def fold_pass(s, store2d):
        def fbody(k, carry):
            v = y_v[pl.ds(k * 16, 16)] + y_v[pl.ds(k * 16 + s, 16)]
            if store2d:
                y2d[k, :] = v
            else:
                y_v[pl.ds(k * 16, 16)] = v
            return carry
        lax.fori_loop(0, R, fbody, 0, unroll=4)

    fold_pass(8, False)
    fold_pass(4, False)
    fold_pass(2, False)
    fold_pass(1, True)
    sid = lax.axis_index("s")
    pltpu.sync_copy(y2d.at[:, 0], ysh.at[sid])
    pltpu.sync_copy(ysh.at[sid], out_hbm.at[pl.ds(wid * R, R)])


@functools.partial(
    pl.kernel,
    out_type=jax.ShapeDtypeStruct((N_SC,), jnp.float32),
    mesh=plsc.VectorSubcoreMesh(core_axis_name="c", subcore_axis_name="s", num_cores=1),
    scratch_types=[
        pltpu.VMEM((M,), jnp.float32),
        pltpu.VMEM((BR, M), jnp.float32),
        pltpu.VMEM((BR, M), jnp.float32),
        pltpu.VMEM((R * 16 + 16,), jnp.float32),
        pltpu.VMEM((R, 16), jnp.float32),
        pltpu.VMEM_SHARED((NS, R), jnp.float32),
        pltpu.SemaphoreType.DMA,
        pltpu.SemaphoreType.DMA,
    ],
)
def _sc_mv(a_hbm, x_hbm, out_hbm, *rest):
    _sc_body(a_hbm, x_hbm, out_hbm, *rest)


def _tc_mv_body(a_ref, x_ref, o_ref):
    o_ref[...] = jax.lax.dot_general(
        a_ref[...], x_ref[...],
        (((1,), (0,)), ((), ())),
        preferred_element_type=jnp.float32)


_tc_mv = pl.pallas_call(
    _tc_mv_body,
    grid=(N_TC // TC_BLK,),
    in_specs=[
        pl.BlockSpec((TC_BLK, M), lambda i: (i, 0)),
        pl.BlockSpec((M,), lambda i: (0,)),
    ],
    out_specs=pl.BlockSpec((TC_BLK,), lambda i: (i,)),
    out_shape=jax.ShapeDtypeStruct((N_TC,), jnp.float32),
)


def kernel(A, x):
    y_sc = _sc_mv(A, x)
    y_tc = _tc_mv(A, x)
    return jnp.concatenate([y_tc, y_sc])
